# baseline (speedup 1.0000x reference)
import jax
import jax.numpy as jnp
from jax.experimental import pallas as pl


def _copy_body(x_ref, o_ref):
    o_ref[...] = x_ref[...]


def kernel(x):
    n, f = x.shape
    block = 4096
    return pl.pallas_call(
        _copy_body,
        grid=(n // block,),
        in_specs=[pl.BlockSpec((block, f), lambda i: (i, 0))],
        out_specs=pl.BlockSpec((block, f), lambda i: (i, 0)),
        out_shape=jax.ShapeDtypeStruct(x.shape, x.dtype),
    )(x)
